# baseline (speedup 1.0000x reference)
import functools

import jax
import jax.numpy as jnp
from jax import lax
from jax.experimental import pallas as pl
from jax.experimental.pallas import tpu as pltpu
from jax.experimental.pallas import tpu_sc as plsc

_VOCAB = 1000000
_DIM = 64
_LANES = 16

_NW = 32
_IDXW = 128
_K = 4
_ROWS = _K * _IDXW


def _build(n_chunks):
    mesh = plsc.VectorSubcoreMesh(core_axis_name="c", subcore_axis_name="s")

    @functools.partial(
        pl.kernel,
        out_type=jax.ShapeDtypeStruct((_NW, n_chunks, _ROWS, _DIM), jnp.float32),
        mesh=mesh,
        scratch_types=[
            pltpu.VMEM((_K, _IDXW), jnp.int32),
            pltpu.VMEM((_ROWS, _DIM), jnp.float32),
            pltpu.VMEM((_LANES,), jnp.float32),
            pltpu.SemaphoreType.DMA,
        ],
        compiler_params=pltpu.CompilerParams(use_tc_tiling_on_sc=False),
    )
    def kern(tok_hbm, table_hbm, scale_hbm, out_hbm, idx_v, rows_v, scale_v, sem):
        wid = lax.axis_index("s") * 2 + lax.axis_index("c")
        pltpu.sync_copy(scale_hbm, scale_v)
        svec = scale_v[...]

        def chunk_body(i, _):
            pltpu.sync_copy(tok_hbm.at[wid, i], idx_v)
            for j in range(_K):
                pltpu.async_copy(
                    table_hbm.at[idx_v.at[j]],
                    rows_v.at[pl.ds(j * _IDXW, _IDXW)],
                    sem,
                )
            for j in range(_K):
                pltpu.make_async_copy(
                    table_hbm.at[idx_v.at[j]],
                    rows_v.at[pl.ds(j * _IDXW, _IDXW)],
                    sem,
                ).wait()

            def scale_body(r, _):
                for c in range(_DIM // _LANES):
                    sl = pl.ds(c * _LANES, _LANES)
                    rows_v[r, sl] = rows_v[r, sl] * svec
                return 0

            lax.fori_loop(0, _ROWS, scale_body, 0)
            pltpu.sync_copy(rows_v, out_hbm.at[wid, i])
            return 0

        lax.fori_loop(0, n_chunks, chunk_body, 0)

    return kern


def kernel(token_ids, embed_weight, scale):
    b, s = token_ids.shape
    n = b * s
    assert n % (_NW * _ROWS) == 0
    n_chunks = n // (_NW * _ROWS)
    tok = token_ids.reshape(_NW, n_chunks, _K, _IDXW).astype(jnp.int32)
    scale_vec = jnp.broadcast_to(scale.astype(jnp.float32), (_LANES,))
    out = _build(n_chunks)(tok, embed_weight, scale_vec)
    return out.reshape(b, s, _DIM)


# speedup vs baseline: 1.1372x; 1.1372x over previous
import functools

import jax
import jax.numpy as jnp
from jax import lax
from jax.experimental import pallas as pl
from jax.experimental.pallas import tpu as pltpu
from jax.experimental.pallas import tpu_sc as plsc

_DIM = 64
_LANES = 16

_NW = 32
_IDXW = 128
_K = 2
_ROWS = _K * _IDXW
_NB = 4
_UNROLL = 8


def _build(n_chunks):
    mesh = plsc.VectorSubcoreMesh(core_axis_name="c", subcore_axis_name="s")

    @functools.partial(
        pl.kernel,
        out_type=jax.ShapeDtypeStruct((_NW, n_chunks, _ROWS, _DIM), jnp.float32),
        mesh=mesh,
        scratch_types=[
            pltpu.VMEM((n_chunks, _K, _IDXW), jnp.int32),
            pltpu.VMEM((_NB, _ROWS, _DIM), jnp.float32),
            pltpu.VMEM((_LANES,), jnp.float32),
            [pltpu.SemaphoreType.DMA for _ in range(_NB)],
            [pltpu.SemaphoreType.DMA for _ in range(_NB)],
        ],
        compiler_params=pltpu.CompilerParams(use_tc_tiling_on_sc=False),
    )
    def kern(tok_hbm, table_hbm, scale_hbm, out_hbm, idx_v, rows_v, scale_v,
             gsems, osems):
        wid = lax.axis_index("s") * 2 + lax.axis_index("c")
        pltpu.sync_copy(scale_hbm, scale_v)
        pltpu.sync_copy(tok_hbm.at[wid], idx_v)
        svec = scale_v[...]

        def fire_gather(i, b):
            for j in range(_K):
                pltpu.async_copy(
                    table_hbm.at[idx_v.at[i, j]],
                    rows_v.at[b, pl.ds(j * _IDXW, _IDXW)],
                    gsems[b],
                )

        def wait_gather(i, b):
            for j in range(_K):
                pltpu.make_async_copy(
                    table_hbm.at[idx_v.at[i, j]],
                    rows_v.at[b, pl.ds(j * _IDXW, _IDXW)],
                    gsems[b],
                ).wait()

        def out_copy(i, b):
            return pltpu.make_async_copy(rows_v.at[b], out_hbm.at[wid, i], osems[b])

        fire_gather(0, 0)
        fire_gather(1, 1)

        def ring_body(g, _):
            for b in range(_NB):
                i = g * _NB + b
                wait_gather(i, b)

                def scale_body(r, _):
                    for u in range(_UNROLL):
                        for c in range(_DIM // _LANES):
                            sl = pl.ds(c * _LANES, _LANES)
                            rows_v[b, r * _UNROLL + u, sl] = (
                                rows_v[b, r * _UNROLL + u, sl] * svec)
                    return 0

                lax.fori_loop(0, _ROWS // _UNROLL, scale_body, 0)
                out_copy(i, b).start()

                b2 = (b + 2) % _NB
                @pl.when(i >= 2)
                def _():
                    out_copy(i - 2, b2).wait()
                @pl.when(i + 2 < n_chunks)
                def _():
                    fire_gather(i + 2, b2)
            return 0

        lax.fori_loop(0, n_chunks // _NB, ring_body, 0)
        out_copy(n_chunks - 2, (n_chunks - 2) % _NB).wait()
        out_copy(n_chunks - 1, (n_chunks - 1) % _NB).wait()

    return kern


def kernel(token_ids, embed_weight, scale):
    b, s = token_ids.shape
    n = b * s
    assert n % (_NW * _ROWS * _NB) == 0
    n_chunks = n // (_NW * _ROWS)
    tok = token_ids.reshape(_NW, n_chunks, _K, _IDXW).astype(jnp.int32)
    scale_vec = jnp.broadcast_to(scale.astype(jnp.float32), (_LANES,))
    out = _build(n_chunks)(tok, embed_weight, scale_vec)
    return out.reshape(b, s, _DIM)
